# Initial kernel scaffold; baseline (speedup 1.0000x reference)
import jax
import jax.numpy as jnp
from jax.experimental import pallas as pl


def kernel(data, embed_table):
    raise NotImplementedError("write your pallas kernel here")


# speedup vs baseline: 5.3418x; 5.3418x over previous
import functools

import jax
import jax.numpy as jnp
from jax import lax
from jax.experimental import pallas as pl
from jax.experimental.pallas import tpu as pltpu
from jax.experimental.pallas import tpu_sc as plsc

NC = 2
NS = 16
NW = NC * NS

IDX_VECS_PER_CHUNK = 8
CHUNK = IDX_VECS_PER_CHUNK * 128


def _gather_kernel(n_chunks, idx_hbm, table_hbm, out_hbm, idx_v, rows_v, sem):
    wid = lax.axis_index("s") * NC + lax.axis_index("c")

    def chunk_body(i, carry):
        row0 = (wid * n_chunks + i) * IDX_VECS_PER_CHUNK
        pltpu.sync_copy(idx_hbm.at[pl.ds(row0, IDX_VECS_PER_CHUNK)], idx_v)
        cps = [
            pltpu.async_copy(
                table_hbm.at[idx_v.at[j]],
                rows_v.at[pl.ds(j * 128, 128)],
                sem,
            )
            for j in range(IDX_VECS_PER_CHUNK)
        ]
        for cp in cps:
            cp.wait()
        out0 = (wid * n_chunks + i) * CHUNK
        pltpu.sync_copy(rows_v, out_hbm.at[pl.ds(out0, CHUNK)])
        return carry

    lax.fori_loop(0, n_chunks, chunk_body, 0)


def kernel(data, embed_table):
    b, l, s = data.shape
    n = b * l * s
    d = embed_table.shape[1]
    assert n % (NW * CHUNK) == 0
    n_chunks = n // (NW * CHUNK)

    idx_2d = data.reshape(n // 128, 128)

    mesh = plsc.VectorSubcoreMesh(
        core_axis_name="c", subcore_axis_name="s",
        num_cores=NC, num_subcores=NS,
    )
    run = pl.kernel(
        functools.partial(_gather_kernel, n_chunks),
        out_type=jax.ShapeDtypeStruct((n, d), jnp.float32),
        mesh=mesh,
        scratch_types=[
            pltpu.VMEM((IDX_VECS_PER_CHUNK, 128), jnp.int32),
            pltpu.VMEM((CHUNK, d), jnp.float32),
            pltpu.SemaphoreType.DMA,
        ],
        compiler_params=pltpu.CompilerParams(use_tc_tiling_on_sc=False),
    )
    out = run(idx_2d, embed_table)
    return out.reshape(b, l, s * d)


# speedup vs baseline: 9.7581x; 1.8267x over previous
import functools

import jax
import jax.numpy as jnp
from jax import lax
from jax.experimental import pallas as pl
from jax.experimental.pallas import tpu as pltpu
from jax.experimental.pallas import tpu_sc as plsc

NC = 2
NS = 16
NW = NC * NS

IDX_VECS_PER_CHUNK = 8
CHUNK = IDX_VECS_PER_CHUNK * 128


def _gather_kernel(n_chunks, idx_hbm, table_hbm, out_hbm, table_v, idx_v,
                   rows_v, sem):
    sid = lax.axis_index("s")
    wid = sid * NC + lax.axis_index("c")

    @pl.when(sid == 0)
    def _():
        pltpu.sync_copy(table_hbm, table_v)

    plsc.subcore_barrier()

    def chunk_body(i, carry):
        row0 = (wid * n_chunks + i) * IDX_VECS_PER_CHUNK
        pltpu.sync_copy(idx_hbm.at[pl.ds(row0, IDX_VECS_PER_CHUNK)], idx_v)
        cps = [
            pltpu.async_copy(
                table_v.at[idx_v.at[j]],
                rows_v.at[pl.ds(j * 128, 128)],
                sem,
            )
            for j in range(IDX_VECS_PER_CHUNK)
        ]
        for cp in cps:
            cp.wait()
        out0 = (wid * n_chunks + i) * CHUNK
        pltpu.sync_copy(rows_v, out_hbm.at[pl.ds(out0, CHUNK)])
        return carry

    lax.fori_loop(0, n_chunks, chunk_body, 0)


def kernel(data, embed_table):
    b, l, s = data.shape
    n = b * l * s
    d = embed_table.shape[1]
    assert n % (NW * CHUNK) == 0
    n_chunks = n // (NW * CHUNK)

    idx_2d = data.reshape(n // 128, 128)

    mesh = plsc.VectorSubcoreMesh(
        core_axis_name="c", subcore_axis_name="s",
        num_cores=NC, num_subcores=NS,
    )
    run = pl.kernel(
        functools.partial(_gather_kernel, n_chunks),
        out_type=jax.ShapeDtypeStruct((n, d), jnp.float32),
        mesh=mesh,
        scratch_types=[
            pltpu.VMEM_SHARED((133, d), jnp.float32),
            pltpu.VMEM((IDX_VECS_PER_CHUNK, 128), jnp.int32),
            pltpu.VMEM((CHUNK, d), jnp.float32),
            pltpu.SemaphoreType.DMA,
        ],
        compiler_params=pltpu.CompilerParams(use_tc_tiling_on_sc=False),
    )
    out = run(idx_2d, embed_table)
    return out.reshape(b, l, s * d)


# speedup vs baseline: 11.0380x; 1.1312x over previous
import functools

import jax
import jax.numpy as jnp
from jax import lax
from jax.experimental import pallas as pl
from jax.experimental.pallas import tpu as pltpu
from jax.experimental.pallas import tpu_sc as plsc

NC = 2
NS = 16
NW = NC * NS

NBUF = 4
VECS = 5
CHUNK = VECS * 128


def _gather_kernel(n_chunks, d, idx_hbm, table_hbm, out_hbm, table_v, idx_v,
                   rows_v, isem, gsem, ssem):
    sid = lax.axis_index("s")
    wid = sid * NC + lax.axis_index("c")
    chunk0 = wid * n_chunks
    n_groups = n_chunks // NBUF

    @pl.when(sid == 0)
    def _():
        pltpu.sync_copy(table_hbm, table_v)

    plsc.subcore_barrier()

    def start_idx(i, b):
        pltpu.async_copy(
            idx_hbm.at[pl.ds((chunk0 + i) * VECS, VECS)],
            idx_v.at[b], isem.at[b])

    def wait_idx(b):
        pltpu.make_async_copy(
            idx_hbm.at[pl.ds(0, VECS)], idx_v.at[b], isem.at[b]).wait()

    def fire_gathers(b):
        for j in range(VECS):
            pltpu.async_copy(
                table_v.at[idx_v.at[b].at[j]],
                rows_v.at[b].at[pl.ds(j * 128, 128)],
                gsem.at[b])

    def wait_gathers(b):
        pltpu.make_async_copy(
            out_hbm.at[pl.ds(0, CHUNK)], rows_v.at[b], gsem.at[b]).wait()

    def start_store(i, b):
        pltpu.async_copy(
            rows_v.at[b], out_hbm.at[pl.ds((chunk0 + i) * CHUNK, CHUNK)],
            ssem.at[b])

    def wait_store(b):
        pltpu.make_async_copy(
            rows_v.at[b], out_hbm.at[pl.ds(0, CHUNK)], ssem.at[b]).wait()

    for b in range(NBUF):
        start_idx(b, b)
    for b in range(NBUF):
        wait_idx(b)
        fire_gathers(b)

    def group_body(g, carry):
        i0 = g * NBUF
        for b in range(NBUF):
            wait_gathers(b)
            start_store(i0 + b, b)
            start_idx(i0 + NBUF + b, b)
        for b in range(NBUF):
            wait_store(b)
            wait_idx(b)
            fire_gathers(b)
        return carry

    lax.fori_loop(0, n_groups - 1, group_body, 0)

    i0 = (n_groups - 1) * NBUF
    for b in range(NBUF):
        wait_gathers(b)
        start_store(i0 + b, b)
    for b in range(NBUF):
        wait_store(b)


def kernel(data, embed_table):
    b, l, s = data.shape
    n = b * l * s
    d = embed_table.shape[1]
    assert n % (NW * CHUNK * NBUF) == 0
    n_chunks = n // (NW * CHUNK)

    idx_2d = data.reshape(n // 128, 128)

    mesh = plsc.VectorSubcoreMesh(
        core_axis_name="c", subcore_axis_name="s",
        num_cores=NC, num_subcores=NS,
    )
    run = pl.kernel(
        functools.partial(_gather_kernel, n_chunks, d),
        out_type=jax.ShapeDtypeStruct((n, d), jnp.float32),
        mesh=mesh,
        scratch_types=[
            pltpu.VMEM_SHARED((133, d), jnp.float32),
            pltpu.VMEM((NBUF, VECS, 128), jnp.int32),
            pltpu.VMEM((NBUF, CHUNK, d), jnp.float32),
            pltpu.SemaphoreType.DMA((NBUF,)),
            pltpu.SemaphoreType.DMA((NBUF,)),
            pltpu.SemaphoreType.DMA((NBUF,)),
        ],
        compiler_params=pltpu.CompilerParams(use_tc_tiling_on_sc=False),
    )
    out = run(idx_2d, embed_table)
    return out.reshape(b, l, s * d)


# speedup vs baseline: 54.4848x; 4.9361x over previous
import functools

import jax
import jax.numpy as jnp
from jax import lax
from jax.experimental import pallas as pl
from jax.experimental.pallas import tpu as pltpu
from jax.experimental.pallas import tpu_sc as plsc

NC = 2
NS = 16
NW = NC * NS

NBUF = 4
TILE = 128
SLOTS = 4


def _gather_kernel(n_tiles, d, idx_hbm, table_hbm, out_hbm, table_v, idx_v,
                   rows_v, isem, gsem, ssem):
    sid = lax.axis_index("s")
    wid = sid * NC + lax.axis_index("c")
    row0 = wid * n_tiles * TILE
    n_groups = n_tiles // NBUF

    @pl.when(sid == 0)
    def _():
        pltpu.sync_copy(table_hbm, table_v)

    plsc.subcore_barrier()

    def start_idx(i, b):
        pltpu.async_copy(
            idx_hbm.at[:, pl.ds(row0 + i * TILE, TILE)],
            idx_v.at[b], isem.at[b])

    def wait_idx(b):
        pltpu.make_async_copy(
            idx_hbm.at[:, pl.ds(0, TILE)], idx_v.at[b], isem.at[b]).wait()

    def fire_gathers(b):
        for q in range(SLOTS):
            pltpu.async_copy(
                table_v.at[idx_v.at[b].at[q]],
                rows_v.at[b].at[q],
                gsem.at[b])

    def wait_gathers(b):
        pltpu.make_async_copy(
            out_hbm.at[pl.ds(0, SLOTS * TILE), pl.ds(0, d)], rows_v.at[b],
            gsem.at[b]).wait()

    def start_store(i, b):
        for q in range(SLOTS):
            pltpu.async_copy(
                rows_v.at[b].at[q],
                out_hbm.at[pl.ds(row0 + i * TILE, TILE), pl.ds(q * d, d)],
                ssem.at[b])

    def wait_store(b):
        pltpu.make_async_copy(
            out_hbm.at[pl.ds(0, SLOTS * TILE), pl.ds(0, d)], rows_v.at[b],
            ssem.at[b]).wait()

    for b in range(NBUF):
        start_idx(b, b)
    for b in range(NBUF):
        wait_idx(b)
        fire_gathers(b)

    def group_body(g, carry):
        i0 = g * NBUF
        for b in range(NBUF):
            wait_gathers(b)
            start_store(i0 + b, b)
            start_idx(i0 + NBUF + b, b)
        for b in range(NBUF):
            wait_store(b)
            wait_idx(b)
            fire_gathers(b)
        return carry

    lax.fori_loop(0, n_groups - 1, group_body, 0)

    i0 = (n_groups - 1) * NBUF
    for b in range(NBUF):
        wait_gathers(b)
        start_store(i0 + b, b)
    for b in range(NBUF):
        wait_store(b)


def kernel(data, embed_table):
    b, l, s = data.shape
    n_rows = b * l
    d = embed_table.shape[1]
    assert s == SLOTS and s * d == 128
    assert n_rows % (NW * TILE * NBUF) == 0
    n_tiles = n_rows // (NW * TILE)

    idx_t = data.reshape(n_rows, s).T

    mesh = plsc.VectorSubcoreMesh(
        core_axis_name="c", subcore_axis_name="s",
        num_cores=NC, num_subcores=NS,
    )
    run = pl.kernel(
        functools.partial(_gather_kernel, n_tiles, d),
        out_type=jax.ShapeDtypeStruct((n_rows, s * d), jnp.float32),
        mesh=mesh,
        scratch_types=[
            pltpu.VMEM_SHARED((133, d), jnp.float32),
            pltpu.VMEM((NBUF, SLOTS, TILE), jnp.int32),
            pltpu.VMEM((NBUF, SLOTS, TILE, d), jnp.float32),
            pltpu.SemaphoreType.DMA((NBUF,)),
            pltpu.SemaphoreType.DMA((NBUF,)),
            pltpu.SemaphoreType.DMA((NBUF,)),
        ],
        compiler_params=pltpu.CompilerParams(use_tc_tiling_on_sc=False),
    )
    out = run(idx_t, embed_table)
    return out.reshape(b, l, s * d)
